# Initial kernel scaffold; baseline (speedup 1.0000x reference)
import functools

import jax
import jax.numpy as jnp
from jax import lax
from jax.experimental import pallas as pl
from jax.experimental.pallas import tpu as pltpu
from jax.experimental.pallas import tpu_sc as plsc

B = 16384
D = 129
NFULL = 8
NC = 2
NS = 16
NW = NC * NS
BPW = B // NW
C = 128
NCHUNK = BPW // C

_mesh = plsc.VectorSubcoreMesh(core_axis_name="c", subcore_axis_name="s")


@functools.partial(
    pl.kernel,
    mesh=_mesh,
    out_type=jax.ShapeDtypeStruct((B,), jnp.float32),
    scratch_types=[
        pltpu.VMEM((C,), jnp.int32),
        pltpu.VMEM((C,), jnp.int32),
        pltpu.VMEM((C, D), jnp.float32),
        pltpu.VMEM((C, D), jnp.float32),
        pltpu.VMEM((C,), jnp.float32),
        pltpu.SemaphoreType.DMA,
        pltpu.SemaphoreType.DMA,
    ],
)
def _sc_dot(users_hbm, items_hbm, uemb_hbm, iemb_hbm, out_hbm,
            uidx, iidx, urows, irows, outc, sem_u, sem_i):
    wid = lax.axis_index("s") * NC + lax.axis_index("c")
    base = wid * BPW
    last_lane = jnp.arange(16, dtype=jnp.int32) == 15

    def chunk_body(ci, carry):
        off = pl.multiple_of(base + ci * C, C)
        pltpu.sync_copy(users_hbm.at[pl.ds(off, C)], uidx)
        pltpu.sync_copy(items_hbm.at[pl.ds(off, C)], iidx)
        cu = pltpu.async_copy(uemb_hbm.at[uidx], urows, sem_u)
        cit = pltpu.async_copy(iemb_hbm.at[iidx], irows, sem_i)
        cu.wait()
        cit.wait()

        def row_body(r, rcarry):
            acc = urows[r, pl.ds(0, 16)] * irows[r, pl.ds(0, 16)]
            for j in range(1, NFULL):
                acc = acc + urows[r, pl.ds(j * 16, 16)] * irows[r, pl.ds(j * 16, 16)]
            tail = urows[r, pl.ds(D - 16, 16)] * irows[r, pl.ds(D - 16, 16)]
            acc = acc + jnp.where(last_lane, tail, 0.0)
            outc[r] = jnp.sum(acc)
            return rcarry

        lax.fori_loop(0, C, row_body, 0)
        pltpu.sync_copy(outc, out_hbm.at[pl.ds(off, C)])
        return carry

    lax.fori_loop(0, NCHUNK, chunk_body, 0)


def kernel(users, items, user_emb, item_emb):
    return _sc_dot(users, items, user_emb, item_emb)


# speedup vs baseline: 5.6804x; 5.6804x over previous
import functools

import jax
import jax.numpy as jnp
from jax import lax
from jax.experimental import pallas as pl
from jax.experimental.pallas import tpu as pltpu
from jax.experimental.pallas import tpu_sc as plsc

B = 16384
D = 129
DM = 128
NFULL = DM // 16
NC = 2
NS = 16
NW = NC * NS
BPW = B // NW
C = 128
NCHUNK = BPW // C
G = 16

_mesh = plsc.VectorSubcoreMesh(core_axis_name="c", subcore_axis_name="s")


@functools.partial(
    pl.kernel,
    mesh=_mesh,
    out_type=jax.ShapeDtypeStruct((B,), jnp.float32),
    compiler_params=pltpu.CompilerParams(needs_layout_passes=False),
    scratch_types=[
        pltpu.VMEM((C,), jnp.int32),
        pltpu.VMEM((C,), jnp.int32),
        pltpu.VMEM((C, DM), jnp.float32),
        pltpu.VMEM((C, DM), jnp.float32),
        pltpu.VMEM((C,), jnp.float32),
        pltpu.VMEM((C,), jnp.float32),
        pltpu.VMEM((C,), jnp.float32),
        pltpu.SemaphoreType.DMA,
        pltpu.SemaphoreType.DMA,
    ],
)
def _sc_dot(users_hbm, items_hbm, ub_hbm, ib_hbm, uemb_hbm, iemb_hbm, out_hbm,
            uidx, iidx, urows, irows, ubv, ibv, outc, sem_u, sem_i):
    wid = lax.axis_index("s") * NC + lax.axis_index("c")
    base = wid * BPW
    lane = jnp.arange(16, dtype=jnp.int32)

    def chunk_body(ci, carry):
        off = pl.multiple_of(base + ci * C, C)
        pltpu.sync_copy(users_hbm.at[pl.ds(off, C)], uidx)
        pltpu.sync_copy(items_hbm.at[pl.ds(off, C)], iidx)
        cu = pltpu.async_copy(uemb_hbm.at[uidx, pl.ds(0, DM)], urows, sem_u)
        cit = pltpu.async_copy(iemb_hbm.at[iidx, pl.ds(0, DM)], irows, sem_i)
        pltpu.sync_copy(ub_hbm.at[pl.ds(off, C)], ubv)
        pltpu.sync_copy(ib_hbm.at[pl.ds(off, C)], ibv)
        cu.wait()
        cit.wait()

        def group_body(g, gcarry):
            gbase = g * G
            res = ubv[pl.ds(gbase, 16)] * ibv[pl.ds(gbase, 16)]
            for k in range(G):
                r = gbase + k
                acc = urows[r, pl.ds(0, 16)] * irows[r, pl.ds(0, 16)]
                for j in range(1, NFULL):
                    acc = acc + urows[r, pl.ds(j * 16, 16)] * irows[r, pl.ds(j * 16, 16)]
                res = jnp.where(lane == k, res + jnp.sum(acc), res)
            outc[pl.ds(gbase, 16)] = res
            return gcarry

        lax.fori_loop(0, C // G, group_body, 0)
        pltpu.sync_copy(outc, out_hbm.at[pl.ds(off, C)])
        return carry

    lax.fori_loop(0, NCHUNK, chunk_body, 0)


def kernel(users, items, user_emb, item_emb):
    u_bias = user_emb[users, DM]
    i_bias = item_emb[items, DM]
    return _sc_dot(users, items, u_bias, i_bias, user_emb, item_emb)


# speedup vs baseline: 9.3364x; 1.6436x over previous
import functools

import jax
import jax.numpy as jnp
from jax import lax
from jax.experimental import pallas as pl
from jax.experimental.pallas import tpu as pltpu
from jax.experimental.pallas import tpu_sc as plsc

B = 16384
D = 129
W = 128
NB = 1000000
TAIL = (NB // W) * W
TW = NB - TAIL
NC, NS = 2, 16
NW = NC * NS
BPW = B // NW
R = 2
STEPS = BPW // R

_mesh = plsc.VectorSubcoreMesh(core_axis_name="c", subcore_axis_name="s")


@functools.partial(
    pl.kernel,
    mesh=_mesh,
    out_type=jax.ShapeDtypeStruct((B,), jnp.float32),
    compiler_params=pltpu.CompilerParams(needs_layout_passes=False),
    scratch_types=[
        pltpu.VMEM((BPW,), jnp.int32),
        pltpu.VMEM((BPW,), jnp.int32),
        pltpu.VMEM((D, W), jnp.float32),
        pltpu.VMEM((D, W), jnp.float32),
        pltpu.VMEM((D, W), jnp.float32),
        pltpu.VMEM((D, W), jnp.float32),
        pltpu.VMEM((D, TW), jnp.float32),
        pltpu.VMEM((BPW,), jnp.float32),
        pltpu.SemaphoreType.DMA,
        pltpu.SemaphoreType.DMA,
        pltpu.SemaphoreType.DMA,
        pltpu.SemaphoreType.DMA,
    ],
)
def _sc_dot2(users_hbm, items_hbm, uT_hbm, iT_hbm, out_hbm,
             uv, iv, ublk0, ublk1, iblk0, iblk1, tailb, outc,
             sem_u0, sem_u1, sem_i0, sem_i1):
    wid = lax.axis_index("s") * NC + lax.axis_index("c")
    base = wid * BPW
    lane = jnp.arange(16, dtype=jnp.int32)
    d128 = jnp.full((16,), 128, jnp.int32)

    pltpu.sync_copy(users_hbm.at[pl.ds(base, BPW)], uv)
    pltpu.sync_copy(items_hbm.at[pl.ds(base, BPW)], iv)

    sem_u = (sem_u0, sem_u1)
    sem_i = (sem_i0, sem_i1)
    ublk = (ublk0, ublk1)
    iblk = (iblk0, iblk1)

    def sread(ref, b):
        return plsc.load_gather(ref, [jnp.zeros((16,), jnp.int32) + b])[0]

    def pair_normal(b):
        ru = sread(uv, b)
        ri = sread(iv, b)
        return (ru < TAIL) & (ri < TAIL), ru, ri

    def issue(b, k):
        ok, ru, ri = pair_normal(b)

        @pl.when(ok)
        def _():
            cu = pl.multiple_of((ru // W) * W, W)
            ci = pl.multiple_of((ri // W) * W, W)
            pltpu.async_copy(uT_hbm.at[:, pl.ds(cu, W)], ublk[k], sem_u[k])
            pltpu.async_copy(iT_hbm.at[:, pl.ds(ci, W)], iblk[k], sem_i[k])

    def extract(r, blk):
        col = jnp.zeros((16,), jnp.int32) + (r % W)
        pieces = []
        for j in range(8):
            pieces.append(plsc.load_gather(blk, [lane + j * 16, col]))
        pieces.append(plsc.load_gather(blk, [d128, col]))
        return pieces

    def dot(up, ip):
        acc = up[0] * ip[0]
        for j in range(1, 8):
            acc = acc + up[j] * ip[j]
        acc = acc + jnp.where(lane == 0, up[8] * ip[8], 0.0)
        return jnp.sum(acc)

    for k in range(R):
        issue(k, k)

    def step(t, res):
        for k in range(R):
            b = t * R + k
            ok, ru, ri = pair_normal(b)

            @pl.when(ok)
            def _():
                pltpu.make_async_copy(
                    uT_hbm.at[:, pl.ds(0, W)], ublk[k], sem_u[k]).wait()
                pltpu.make_async_copy(
                    iT_hbm.at[:, pl.ds(0, W)], iblk[k], sem_i[k]).wait()

            up = extract(ru, ublk[k])
            ip = extract(ri, iblk[k])
            s = dot(up, ip)

            @pl.when(b + R < BPW)
            def _():
                issue(b + R, k)

            res = jnp.where(ok & (lane == (b % 16)), s, res)

        @pl.when(t % 8 == 7)
        def _():
            outc[pl.ds((t // 8) * 16, 16)] = res

        return res

    lax.fori_loop(0, STEPS, step, jnp.zeros((16,), jnp.float32))

    def fetch_one(r, tbl):
        @pl.when(r < TAIL)
        def _():
            cb = pl.multiple_of((r // W) * W, W)
            pltpu.sync_copy(tbl.at[:, pl.ds(cb, W)], ublk0)

        @pl.when(r >= TAIL)
        def _():
            pltpu.sync_copy(tbl.at[:, pl.ds(TAIL, TW)], tailb)

        colm = jnp.zeros((16,), jnp.int32) + (r % W)
        colt = jnp.clip(jnp.zeros((16,), jnp.int32) + (r - TAIL), 0, TW - 1)
        pieces = []
        for j in range(8):
            m = plsc.load_gather(ublk0, [lane + j * 16, colm])
            tl = plsc.load_gather(tailb, [lane + j * 16, colt])
            pieces.append(jnp.where(r < TAIL, m, tl))
        m = plsc.load_gather(ublk0, [d128, colm])
        tl = plsc.load_gather(tailb, [d128, colt])
        pieces.append(jnp.where(r < TAIL, m, tl))
        return pieces

    def ep_step(b, carry):
        ok, ru, ri = pair_normal(b)

        @pl.when(jnp.logical_not(ok))
        def _():
            up = fetch_one(ru, uT_hbm)
            ip = fetch_one(ri, iT_hbm)
            s = dot(up, ip)
            gb = (b // 16) * 16
            old = outc[pl.ds(gb, 16)]
            outc[pl.ds(gb, 16)] = jnp.where(lane == (b % 16), s, old)

        return carry

    lax.fori_loop(0, BPW, ep_step, 0)
    pltpu.sync_copy(outc, out_hbm.at[pl.ds(base, BPW)])


def kernel(users, items, user_emb, item_emb):
    return _sc_dot2(users, items, user_emb.T, item_emb.T)
